# baseline (speedup 1.0000x reference)
import jax
import jax.numpy as jnp
from jax.experimental import pallas as pl
from jax.experimental.pallas import tpu as pltpu

_N_E = 1024
_E_DIM = 256
_BETA = 0.25
_N_ROWS = 16384
_PX = 1024
_GRID = _N_ROWS // _PX


def _vq_kernel(z_ref, w_ref, esq_ref,
               enc_ref, zq_ref, idx0_ref, idx1_ref, loss_ref, perp_ref,
               counts_ref, lsum_ref):
    i = pl.program_id(0)
    zb = z_ref[0]
    w = w_ref[...]
    m = jax.lax.dot_general(w, zb, (((1,), (0,)), ((), ())))
    z_sq = jnp.sum(zb * zb, axis=0, keepdims=True)
    d = (z_sq + esq_ref[...]) - 2.0 * m

    iota_e = jax.lax.broadcasted_iota(jnp.int32, d.shape, 0)
    dmin = jnp.min(d, axis=0, keepdims=True)
    idx0 = jnp.min(jnp.where(d == dmin, iota_e, _N_E), axis=0, keepdims=True)
    hot0 = iota_e == idx0
    d2 = jnp.where(hot0, jnp.inf, d)
    dmin2 = jnp.min(d2, axis=0, keepdims=True)
    idx1 = jnp.min(jnp.where(d2 == dmin2, iota_e, _N_E), axis=0, keepdims=True)

    idx0_ref[...] = idx0.reshape(1, 1, _PX)
    idx1_ref[...] = idx1.reshape(1, 1, _PX)

    idx0_s = jnp.transpose(idx0)
    iota_l = jax.lax.broadcasted_iota(jnp.int32, (_PX, _N_E), 1)
    enc = (iota_l == idx0_s).astype(jnp.float32)
    enc_ref[...] = enc

    hot0f = hot0.astype(jnp.float32)
    zq = jax.lax.dot_general(w, hot0f, (((0,), (0,)), ((), ())))
    zq_ref[0] = zq

    diff = zq - zb
    part = jnp.sum(diff * diff).reshape(1, 1)
    csum = jnp.sum(hot0f, axis=1, keepdims=True)

    @pl.when(i == 0)
    def _init():
        lsum_ref[...] = jnp.zeros_like(lsum_ref)
        counts_ref[...] = jnp.zeros_like(counts_ref)

    lsum_ref[...] += part
    counts_ref[...] += csum

    @pl.when(i == pl.num_programs(0) - 1)
    def _fini():
        m_ = lsum_ref[...] / (_N_ROWS * _E_DIM)
        loss_ref[...] = m_ + _BETA * m_
        p = counts_ref[...] * (1.0 / _N_ROWS)
        ent = jnp.sum(p * jnp.log(p + 1e-10)).reshape(1, 1)
        perp_ref[...] = jnp.exp(-ent)


def kernel(z, W):
    z3 = z.reshape(16, _E_DIM, _PX)
    e_sq = jnp.sum(W ** 2, axis=1).reshape(_N_E, 1)

    out_shapes = (
        jax.ShapeDtypeStruct((_N_ROWS, _N_E), jnp.float32),
        jax.ShapeDtypeStruct((16, _E_DIM, _PX), jnp.float32),
        jax.ShapeDtypeStruct((16, 1, _PX), jnp.int32),
        jax.ShapeDtypeStruct((16, 1, _PX), jnp.int32),
        jax.ShapeDtypeStruct((1, 1), jnp.float32),
        jax.ShapeDtypeStruct((1, 1), jnp.float32),
    )
    enc, zq, idx0, idx1, loss, perp = pl.pallas_call(
        _vq_kernel,
        grid=(_GRID,),
        in_specs=[
            pl.BlockSpec((1, _E_DIM, _PX), lambda i: (i, 0, 0)),
            pl.BlockSpec((_N_E, _E_DIM), lambda i: (0, 0)),
            pl.BlockSpec((_N_E, 1), lambda i: (0, 0)),
        ],
        out_specs=(
            pl.BlockSpec((_PX, _N_E), lambda i: (i, 0)),
            pl.BlockSpec((1, _E_DIM, _PX), lambda i: (i, 0, 0)),
            pl.BlockSpec((1, 1, _PX), lambda i: (i, 0, 0)),
            pl.BlockSpec((1, 1, _PX), lambda i: (i, 0, 0)),
            pl.BlockSpec((1, 1), lambda i: (0, 0)),
            pl.BlockSpec((1, 1), lambda i: (0, 0)),
        ),
        out_shape=out_shapes,
        scratch_shapes=[
            pltpu.VMEM((_N_E, 1), jnp.float32),
            pltpu.VMEM((1, 1), jnp.float32),
        ],
    )(z3, W, e_sq)

    z_q_out = zq.reshape(16, _E_DIM, 32, 32)
    return (loss.reshape(()), z_q_out, perp.reshape(()), enc,
            idx0.reshape(_N_ROWS, 1), idx1.reshape(_N_ROWS, 1))


# speedup vs baseline: 1.2985x; 1.2985x over previous
import jax
import jax.numpy as jnp
from jax.experimental import pallas as pl
from jax.experimental.pallas import tpu as pltpu

_N_E = 1024
_E_DIM = 256
_BETA = 0.25
_N_ROWS = 16384
_P = 1024
_GRID = _N_ROWS // _P


def _vq_kernel(zf_ref, w_ref, esq_ref,
               enc_ref, zq_ref, idx0_ref, idx1_ref, loss_ref, perp_ref,
               counts_ref, lsum_ref):
    i = pl.program_id(0)
    zfb = zf_ref[...]
    w = w_ref[...]
    ze = jax.lax.dot_general(zfb, w, (((1,), (1,)), ((), ())))
    z_sq = jnp.sum(zfb * zfb, axis=1, keepdims=True)
    d = (z_sq + esq_ref[...]) - 2.0 * ze

    iota = jax.lax.broadcasted_iota(jnp.int32, d.shape, 1)
    dmin = jnp.min(d, axis=1, keepdims=True)
    idx0 = jnp.min(jnp.where(d == dmin, iota, _N_E), axis=1, keepdims=True)
    hit0 = iota == idx0
    d2 = jnp.where(hit0, jnp.inf, d)
    dmin2 = jnp.min(d2, axis=1, keepdims=True)
    idx1 = jnp.min(jnp.where(d2 == dmin2, iota, _N_E), axis=1, keepdims=True)

    enc = hit0.astype(jnp.float32)
    enc_ref[...] = enc
    idx0_ref[...] = idx0
    idx1_ref[...] = idx1

    zq = jax.lax.dot_general(enc, w, (((1,), (0,)), ((), ())))
    zq_ref[...] = zq

    diff = zq - zfb
    part = jnp.sum(diff * diff).reshape(1, 1)
    csum = jnp.sum(enc, axis=0, keepdims=True)

    @pl.when(i == 0)
    def _init():
        lsum_ref[...] = jnp.zeros_like(lsum_ref)
        counts_ref[...] = jnp.zeros_like(counts_ref)

    lsum_ref[...] += part
    counts_ref[...] += csum

    @pl.when(i == pl.num_programs(0) - 1)
    def _fini():
        m = lsum_ref[...] / (_N_ROWS * _E_DIM)
        loss_ref[...] = m + _BETA * m
        p = counts_ref[...] * (1.0 / _N_ROWS)
        ent = jnp.sum(p * jnp.log(p + 1e-10)).reshape(1, 1)
        perp_ref[...] = jnp.exp(-ent)


def kernel(z, W):
    zp = jnp.transpose(z, (0, 2, 3, 1))
    zf = zp.reshape(-1, _E_DIM)
    e_sq = jnp.sum(W ** 2, axis=1).reshape(1, _N_E)

    out_shapes = (
        jax.ShapeDtypeStruct((_N_ROWS, _N_E), jnp.float32),
        jax.ShapeDtypeStruct((_N_ROWS, _E_DIM), jnp.float32),
        jax.ShapeDtypeStruct((_N_ROWS, 1), jnp.int32),
        jax.ShapeDtypeStruct((_N_ROWS, 1), jnp.int32),
        jax.ShapeDtypeStruct((1, 1), jnp.float32),
        jax.ShapeDtypeStruct((1, 1), jnp.float32),
    )
    enc, zq, idx0, idx1, loss, perp = pl.pallas_call(
        _vq_kernel,
        grid=(_GRID,),
        in_specs=[
            pl.BlockSpec((_P, _E_DIM), lambda i: (i, 0)),
            pl.BlockSpec((_N_E, _E_DIM), lambda i: (0, 0)),
            pl.BlockSpec((1, _N_E), lambda i: (0, 0)),
        ],
        out_specs=(
            pl.BlockSpec((_P, _N_E), lambda i: (i, 0)),
            pl.BlockSpec((_P, _E_DIM), lambda i: (i, 0)),
            pl.BlockSpec((_P, 1), lambda i: (i, 0)),
            pl.BlockSpec((_P, 1), lambda i: (i, 0)),
            pl.BlockSpec((1, 1), lambda i: (0, 0)),
            pl.BlockSpec((1, 1), lambda i: (0, 0)),
        ),
        out_shape=out_shapes,
        scratch_shapes=[
            pltpu.VMEM((1, _N_E), jnp.float32),
            pltpu.VMEM((1, 1), jnp.float32),
        ],
    )(zf, W, e_sq)

    z_q_out = jnp.transpose(zq.reshape(16, 32, 32, _E_DIM), (0, 3, 1, 2))
    return (loss.reshape(()), z_q_out, perp.reshape(()), enc, idx0, idx1)


# speedup vs baseline: 1.3220x; 1.0181x over previous
import jax
import jax.numpy as jnp
from jax.experimental import pallas as pl
from jax.experimental.pallas import tpu as pltpu

_N_E = 1024
_E_DIM = 256
_BETA = 0.25
_N_ROWS = 16384
_P = 1024
_GRID = _N_ROWS // _P


def _vq_kernel(zf_ref, w_ref, esq_ref,
               enc_ref, zq_ref, idx0_ref, idx1_ref, loss_ref, perp_ref,
               counts_ref, lsum_ref):
    i = pl.program_id(0)
    zfb = zf_ref[...]
    w = w_ref[...]
    ze = jax.lax.dot_general(zfb, w, (((1,), (1,)), ((), ())))
    z_sq = jnp.sum(zfb * zfb, axis=1, keepdims=True)
    d = (z_sq + esq_ref[...]) - 2.0 * ze

    iota_f = jax.lax.broadcasted_iota(jnp.int32, d.shape, 1).astype(jnp.float32)
    dmin = jnp.min(d, axis=1, keepdims=True)
    idx0_f = jnp.min(jnp.where(d == dmin, iota_f, float(_N_E)),
                     axis=1, keepdims=True)
    hit0 = iota_f == idx0_f
    dmin2 = jnp.min(jnp.where(hit0, jnp.inf, d), axis=1, keepdims=True)
    idx1_f = jnp.min(
        jnp.where((d == dmin2) & jnp.logical_not(hit0), iota_f, float(_N_E)),
        axis=1, keepdims=True)

    enc = hit0.astype(jnp.float32)
    enc_ref[...] = enc
    idx0_ref[...] = idx0_f.astype(jnp.int32)
    idx1_ref[...] = idx1_f.astype(jnp.int32)

    zq = jax.lax.dot_general(enc, w, (((1,), (0,)), ((), ())))
    zq_ref[...] = zq

    diff = zq - zfb
    part = jnp.sum(diff * diff).reshape(1, 1)
    ones_row = jnp.ones((1, _P), dtype=jnp.float32)
    csum = jax.lax.dot_general(ones_row, enc, (((1,), (0,)), ((), ())))

    @pl.when(i == 0)
    def _init():
        lsum_ref[...] = jnp.zeros_like(lsum_ref)
        counts_ref[...] = jnp.zeros_like(counts_ref)

    lsum_ref[...] += part
    counts_ref[...] += csum

    @pl.when(i == pl.num_programs(0) - 1)
    def _fini():
        m = lsum_ref[...] / (_N_ROWS * _E_DIM)
        loss_ref[...] = m + _BETA * m
        p = counts_ref[...] * (1.0 / _N_ROWS)
        ent = jnp.sum(p * jnp.log(p + 1e-10)).reshape(1, 1)
        perp_ref[...] = jnp.exp(-ent)


def kernel(z, W):
    zp = jnp.transpose(z, (0, 2, 3, 1))
    zf = zp.reshape(-1, _E_DIM)
    e_sq = jnp.sum(W ** 2, axis=1).reshape(1, _N_E)

    out_shapes = (
        jax.ShapeDtypeStruct((_N_ROWS, _N_E), jnp.float32),
        jax.ShapeDtypeStruct((_N_ROWS, _E_DIM), jnp.float32),
        jax.ShapeDtypeStruct((_N_ROWS, 1), jnp.int32),
        jax.ShapeDtypeStruct((_N_ROWS, 1), jnp.int32),
        jax.ShapeDtypeStruct((1, 1), jnp.float32),
        jax.ShapeDtypeStruct((1, 1), jnp.float32),
    )
    enc, zq, idx0, idx1, loss, perp = pl.pallas_call(
        _vq_kernel,
        grid=(_GRID,),
        in_specs=[
            pl.BlockSpec((_P, _E_DIM), lambda i: (i, 0)),
            pl.BlockSpec((_N_E, _E_DIM), lambda i: (0, 0)),
            pl.BlockSpec((1, _N_E), lambda i: (0, 0)),
        ],
        out_specs=(
            pl.BlockSpec((_P, _N_E), lambda i: (i, 0)),
            pl.BlockSpec((_P, _E_DIM), lambda i: (i, 0)),
            pl.BlockSpec((_P, 1), lambda i: (i, 0)),
            pl.BlockSpec((_P, 1), lambda i: (i, 0)),
            pl.BlockSpec((1, 1), lambda i: (0, 0)),
            pl.BlockSpec((1, 1), lambda i: (0, 0)),
        ),
        out_shape=out_shapes,
        scratch_shapes=[
            pltpu.VMEM((1, _N_E), jnp.float32),
            pltpu.VMEM((1, 1), jnp.float32),
        ],
    )(zf, W, e_sq)

    z_q_out = jnp.transpose(zq.reshape(16, 32, 32, _E_DIM), (0, 3, 1, 2))
    return (loss.reshape(()), z_q_out, perp.reshape(()), enc, idx0, idx1)


# speedup vs baseline: 1.4695x; 1.1115x over previous
import jax
import jax.numpy as jnp
from jax.experimental import pallas as pl
from jax.experimental.pallas import tpu as pltpu

_N_E = 1024
_E_DIM = 256
_BETA = 0.25
_N_ROWS = 16384
_P = 1024
_GRID = _N_ROWS // _P


def _vq_kernel(zf_ref, w_ref,
               enc_ref, zq_ref, idx0_ref, idx1_ref, loss_ref, perp_ref,
               counts_ref, lsum_ref, esq_ref):
    i = pl.program_id(0)

    @pl.when(i == 0)
    def _esq():
        wv = w_ref[...]
        esq_ref[...] = jnp.transpose(
            jnp.sum(wv * wv, axis=1, keepdims=True))

    zfb = zf_ref[...]
    w = w_ref[...]
    ze = jax.lax.dot_general(zfb, w, (((1,), (1,)), ((), ())))
    z_sq = jnp.sum(zfb * zfb, axis=1, keepdims=True)
    d = (z_sq + esq_ref[...]) - 2.0 * ze

    iota_f = jax.lax.broadcasted_iota(jnp.int32, d.shape, 1).astype(jnp.float32)
    dmin = jnp.min(d, axis=1, keepdims=True)
    idx0_f = jnp.min(jnp.where(d == dmin, iota_f, float(_N_E)),
                     axis=1, keepdims=True)
    hit0 = iota_f == idx0_f
    enc = hit0.astype(jnp.float32)
    dbig = d + enc * 1e38
    dmin2 = jnp.min(dbig, axis=1, keepdims=True)
    idx1_f = jnp.min(jnp.where(dbig == dmin2, iota_f, float(_N_E)),
                     axis=1, keepdims=True)
    enc_ref[...] = enc
    idx0_ref[...] = idx0_f.astype(jnp.int32)
    idx1_ref[...] = idx1_f.astype(jnp.int32)

    zq = jax.lax.dot_general(enc, w, (((1,), (0,)), ((), ())))
    zq_ref[...] = zq

    diff = zq - zfb
    part = jnp.sum(diff * diff).reshape(1, 1)
    ones_row = jnp.ones((1, _P), dtype=jnp.float32)
    csum = jax.lax.dot_general(ones_row, enc, (((1,), (0,)), ((), ())))

    @pl.when(i == 0)
    def _init():
        lsum_ref[...] = jnp.zeros_like(lsum_ref)
        counts_ref[...] = jnp.zeros_like(counts_ref)

    lsum_ref[...] += part
    counts_ref[...] += csum

    @pl.when(i == pl.num_programs(0) - 1)
    def _fini():
        m = lsum_ref[...] / (_N_ROWS * _E_DIM)
        loss_ref[...] = m + _BETA * m
        p = counts_ref[...] * (1.0 / _N_ROWS)
        ent = jnp.sum(p * jnp.log(p + 1e-10)).reshape(1, 1)
        perp_ref[...] = jnp.exp(-ent)


def kernel(z, W):
    zp = jnp.transpose(z, (0, 2, 3, 1))
    zf = zp.reshape(-1, _E_DIM)

    out_shapes = (
        jax.ShapeDtypeStruct((_N_ROWS, _N_E), jnp.float32),
        jax.ShapeDtypeStruct((_N_ROWS, _E_DIM), jnp.float32),
        jax.ShapeDtypeStruct((_N_ROWS, 1), jnp.int32),
        jax.ShapeDtypeStruct((_N_ROWS, 1), jnp.int32),
        jax.ShapeDtypeStruct((1, 1), jnp.float32),
        jax.ShapeDtypeStruct((1, 1), jnp.float32),
    )
    enc, zq, idx0, idx1, loss, perp = pl.pallas_call(
        _vq_kernel,
        grid=(_GRID,),
        in_specs=[
            pl.BlockSpec((_P, _E_DIM), lambda i: (i, 0)),
            pl.BlockSpec((_N_E, _E_DIM), lambda i: (0, 0)),
        ],
        out_specs=(
            pl.BlockSpec((_P, _N_E), lambda i: (i, 0)),
            pl.BlockSpec((_P, _E_DIM), lambda i: (i, 0)),
            pl.BlockSpec((_P, 1), lambda i: (i, 0)),
            pl.BlockSpec((_P, 1), lambda i: (i, 0)),
            pl.BlockSpec((1, 1), lambda i: (0, 0)),
            pl.BlockSpec((1, 1), lambda i: (0, 0)),
        ),
        out_shape=out_shapes,
        scratch_shapes=[
            pltpu.VMEM((1, _N_E), jnp.float32),
            pltpu.VMEM((1, 1), jnp.float32),
            pltpu.VMEM((1, _N_E), jnp.float32),
        ],
    )(zf, W)

    z_q_out = jnp.transpose(zq.reshape(16, 32, 32, _E_DIM), (0, 3, 1, 2))
    return (loss.reshape(()), z_q_out, perp.reshape(()), enc, idx0, idx1)


# speedup vs baseline: 1.6775x; 1.1416x over previous
import jax
import jax.numpy as jnp
from jax.experimental import pallas as pl
from jax.experimental.pallas import tpu as pltpu

_N_E = 1024
_E_DIM = 256
_BETA = 0.25
_N_ROWS = 16384
_P = 2048
_GRID = _N_ROWS // _P


def _vq_kernel(zf_ref, w_ref,
               enc_ref, zq_ref, idx0_ref, idx1_ref, loss_ref, perp_ref,
               counts_ref, lsum_ref, esq_ref):
    i = pl.program_id(0)

    @pl.when(i == 0)
    def _esq():
        wv = w_ref[...]
        esq_ref[...] = jnp.transpose(
            jnp.sum(wv * wv, axis=1, keepdims=True))

    zfb = zf_ref[...]
    w = w_ref[...]
    z_sq = jnp.sum(zfb * zfb, axis=1, keepdims=True)
    iota_f = jax.lax.broadcasted_iota(
        jnp.int32, (_P, _N_E), 1).astype(jnp.float32)
    hc = _N_E // 2
    esq = esq_ref[...]
    ze_a = jax.lax.dot_general(zfb, w[:hc], (((1,), (1,)), ((), ())))
    ze_b = jax.lax.dot_general(zfb, w[hc:], (((1,), (1,)), ((), ())))
    d_a = (z_sq + esq[:, :hc]) - 2.0 * ze_a
    d_b = (z_sq + esq[:, hc:]) - 2.0 * ze_b
    d = jnp.concatenate([d_a, d_b], axis=1)

    dmin = jnp.min(d, axis=1, keepdims=True)
    idx0_f = jnp.min(jnp.where(d == dmin, iota_f, float(_N_E)),
                     axis=1, keepdims=True)
    hit0 = iota_f == idx0_f
    enc = hit0.astype(jnp.float32)
    enc_ref[...] = enc
    idx0_ref[...] = idx0_f.astype(jnp.int32)
    h = _P // 2
    zq_parts = [
        jax.lax.dot_general(enc[k * h:(k + 1) * h], w, (((1,), (0,)), ((), ())))
        for k in range(2)
    ]
    ones_row = jnp.ones((1, _P), dtype=jnp.float32)
    csum = jax.lax.dot_general(ones_row, enc, (((1,), (0,)), ((), ())))

    dbig = d + enc * 1e38
    dmin2 = jnp.min(dbig, axis=1, keepdims=True)
    idx1_f = jnp.min(jnp.where(dbig == dmin2, iota_f, float(_N_E)),
                     axis=1, keepdims=True)
    idx1_ref[...] = idx1_f.astype(jnp.int32)

    part = jnp.float32(0.0)
    for k in range(2):
        zq_k = zq_parts[k]
        zq_ref[k * h:(k + 1) * h] = zq_k
        diff_k = zq_k - zfb[k * h:(k + 1) * h]
        part = part + jnp.sum(diff_k * diff_k)
    part = part.reshape(1, 1)

    @pl.when(i == 0)
    def _init():
        lsum_ref[...] = jnp.zeros_like(lsum_ref)
        counts_ref[...] = jnp.zeros_like(counts_ref)

    lsum_ref[...] += part
    counts_ref[...] += csum

    @pl.when(i == pl.num_programs(0) - 1)
    def _fini():
        m = lsum_ref[...] / (_N_ROWS * _E_DIM)
        loss_ref[...] = m + _BETA * m
        p = counts_ref[...] * (1.0 / _N_ROWS)
        ent = jnp.sum(p * jnp.log(p + 1e-10)).reshape(1, 1)
        perp_ref[...] = jnp.exp(-ent)


def kernel(z, W):
    zp = jnp.transpose(z, (0, 2, 3, 1))
    zf = zp.reshape(-1, _E_DIM)

    out_shapes = (
        jax.ShapeDtypeStruct((_N_ROWS, _N_E), jnp.float32),
        jax.ShapeDtypeStruct((_N_ROWS, _E_DIM), jnp.float32),
        jax.ShapeDtypeStruct((_N_ROWS, 1), jnp.int32),
        jax.ShapeDtypeStruct((_N_ROWS, 1), jnp.int32),
        jax.ShapeDtypeStruct((1, 1), jnp.float32),
        jax.ShapeDtypeStruct((1, 1), jnp.float32),
    )
    enc, zq, idx0, idx1, loss, perp = pl.pallas_call(
        _vq_kernel,
        grid=(_GRID,),
        in_specs=[
            pl.BlockSpec((_P, _E_DIM), lambda i: (i, 0)),
            pl.BlockSpec((_N_E, _E_DIM), lambda i: (0, 0)),
        ],
        out_specs=(
            pl.BlockSpec((_P, _N_E), lambda i: (i, 0)),
            pl.BlockSpec((_P, _E_DIM), lambda i: (i, 0)),
            pl.BlockSpec((_P, 1), lambda i: (i, 0)),
            pl.BlockSpec((_P, 1), lambda i: (i, 0)),
            pl.BlockSpec((1, 1), lambda i: (0, 0)),
            pl.BlockSpec((1, 1), lambda i: (0, 0)),
        ),
        out_shape=out_shapes,
        scratch_shapes=[
            pltpu.VMEM((1, _N_E), jnp.float32),
            pltpu.VMEM((1, 1), jnp.float32),
            pltpu.VMEM((1, _N_E), jnp.float32),
        ],
    )(zf, W)

    z_q_out = jnp.transpose(zq.reshape(16, 32, 32, _E_DIM), (0, 3, 1, 2))
    return (loss.reshape(()), z_q_out, perp.reshape(()), enc, idx0, idx1)
